# baseline (speedup 1.0000x reference)
import jax
import jax.numpy as jnp
from jax import lax
from jax.experimental import pallas as pl
from jax.experimental.pallas import tpu as pltpu
from jax.experimental.pallas import tpu_sc as plsc

N_NODES = 10000
N_EDGES = 320000
D_FEAT = 128
H = D_FEAT // 2
NC = 2
NS = 16
EPT = N_EDGES // NS
CHUNK = 128
NCHUNK = 158
EPT_PAD = NCHUNK * CHUNK
N_PAD = 10016
TRASH = N_NODES + 8
RPT = 624
TAIL = N_NODES - NS * RPT
IB = 48


def _gin_body(featc, srcp, dstp, eps16, out, acc, src_v, dst_v, rows,
              eps_v, sem_g, sem_s, sem_i):
    c = lax.axis_index("c")
    s = lax.axis_index("s")

    idx_src = pltpu.async_copy(srcp.at[c, s], src_v, sem_i)
    idx_dst = pltpu.async_copy(dstp.at[s], dst_v, sem_i)

    pltpu.sync_copy(eps16, eps_v)
    scale = eps_v[...] + 1.0

    def init_range(r0, nrows):
        pltpu.sync_copy(featc.at[pl.ds(2 * r0, 2 * nrows)],
                        rows.at[0, pl.ds(0, 2 * nrows)])

        def row_sel(r, carry):
            for j in range(H // 16):
                rows[1, r, pl.ds(j * 16, 16)] = (
                    rows[0, 2 * r + c, pl.ds(j * 16, 16)] * scale)
            return carry

        lax.fori_loop(0, nrows, row_sel, 0)
        pltpu.sync_copy(rows.at[1, pl.ds(0, nrows)],
                        acc.at[pl.ds(r0, nrows)])

    for p in range(RPT // IB):
        init_range(s * RPT + p * IB, IB)

    @pl.when(s == NS - 1)
    def _():
        init_range(NS * RPT, TAIL)

    plsc.subcore_barrier()
    idx_src.wait()
    idx_dst.wait()

    pltpu.async_copy(featc.at[src_v.at[0]], rows.at[0], sem_g.at[0])
    pltpu.async_copy(featc.at[src_v.at[1]], rows.at[1], sem_g.at[1])

    def chunk_body(k, carry):
        b = lax.rem(k, 4)
        fb = lax.rem(k + 2, 4)

        @pl.when(k >= 2)
        def _():
            pltpu.make_async_copy(rows.at[fb], acc.at[dst_v.at[k - 2]],
                                  sem_s.at[fb]).wait()

        @pl.when(k + 2 < NCHUNK)
        def _():
            pltpu.async_copy(featc.at[src_v.at[k + 2]], rows.at[fb],
                             sem_g.at[fb])

        pltpu.make_async_copy(featc.at[src_v.at[k]], rows.at[b],
                              sem_g.at[b]).wait()
        pltpu.async_copy(rows.at[b], acc.at[dst_v.at[k]], sem_s.at[b],
                         add=True)
        return carry

    lax.fori_loop(0, NCHUNK, chunk_body, 0)
    for j in (NCHUNK - 2, NCHUNK - 1):
        pltpu.make_async_copy(rows.at[j % 4], acc.at[dst_v.at[j]],
                              sem_s.at[j % 4]).wait()
    plsc.subcore_barrier()

    pltpu.sync_copy(acc.at[pl.ds(s * RPT, RPT)],
                    out.at[pl.ds(s * RPT, RPT), pl.ds(c * H, H)])

    @pl.when(s == NS - 1)
    def _():
        pltpu.sync_copy(acc.at[pl.ds(NS * RPT, TAIL)],
                        out.at[pl.ds(NS * RPT, TAIL), pl.ds(c * H, H)])


@jax.jit
def kernel(feat, edge_index, eps):
    src = edge_index[0]
    dst = edge_index[1]

    featc = feat.reshape(NC * N_NODES, H)

    pad = EPT_PAD - EPT
    src_t = jnp.pad(src.reshape(NS, EPT), ((0, 0), (0, pad)))
    src_t = (2 * src_t).reshape(NS, NCHUNK, CHUNK)
    srcp = jnp.stack([src_t, src_t + 1])
    dstp = jnp.pad(dst.reshape(NS, EPT), ((0, 0), (0, pad)),
                   constant_values=TRASH).reshape(NS, NCHUNK, CHUNK)

    eps16 = jnp.broadcast_to(eps, (16,))

    mesh = plsc.VectorSubcoreMesh(core_axis_name="c", subcore_axis_name="s")
    out = pl.kernel(
        _gin_body,
        out_type=jax.ShapeDtypeStruct((N_NODES, D_FEAT), jnp.float32),
        mesh=mesh,
        compiler_params=pltpu.CompilerParams(use_tc_tiling_on_sc=False),
        scratch_types=[
            pltpu.VMEM_SHARED((N_PAD, H), jnp.float32),
            pltpu.VMEM((NCHUNK, CHUNK), jnp.int32),
            pltpu.VMEM((NCHUNK, CHUNK), jnp.int32),
            pltpu.VMEM((4, CHUNK, H), jnp.float32),
            pltpu.VMEM((16,), jnp.float32),
            pltpu.SemaphoreType.DMA((4,)),
            pltpu.SemaphoreType.DMA((4,)),
            pltpu.SemaphoreType.DMA,
        ],
    )(featc, srcp, dstp, eps16)
    return out


# speedup vs baseline: 1.2931x; 1.2931x over previous
import jax
import jax.numpy as jnp
from jax import lax
from jax.experimental import pallas as pl
from jax.experimental.pallas import tpu as pltpu
from jax.experimental.pallas import tpu_sc as plsc

N_NODES = 10000
N_EDGES = 320000
D_FEAT = 128
H = D_FEAT // 2
NC = 2
NS = 16
EPT = N_EDGES // NS
CHUNK = 128
NCHUNK = 158
EPT_PAD = NCHUNK * CHUNK
N_PAD = 10016
TRASH = N_NODES + 8
RPT = 624
TAIL = N_NODES - NS * RPT
IB = 208


def _gin_body(featc, srcp, dstp, eps16, out, acc, src_v, dst_v, rows,
              init_buf, eps_v, sem_g, sem_s, sem_i):
    c = lax.axis_index("c")
    s = lax.axis_index("s")

    idx_src = pltpu.async_copy(srcp.at[c, s], src_v, sem_i)
    idx_dst = pltpu.async_copy(dstp.at[s], dst_v, sem_i)

    pltpu.sync_copy(eps16, eps_v)
    scale = eps_v[...] + 1.0

    def init_range(r0, nrows):
        pltpu.sync_copy(featc.at[pl.ds(c * N_NODES + r0, nrows)],
                        init_buf.at[pl.ds(0, nrows)])

        def row_scale(r, carry):
            for j in range(H // 16):
                init_buf[r, pl.ds(j * 16, 16)] = (
                    init_buf[r, pl.ds(j * 16, 16)] * scale)
            return carry

        lax.fori_loop(0, nrows, row_scale, 0)
        pltpu.sync_copy(init_buf.at[pl.ds(0, nrows)],
                        acc.at[pl.ds(r0, nrows)])

    for p in range(RPT // IB):
        init_range(s * RPT + p * IB, IB)

    @pl.when(s == NS - 1)
    def _():
        init_range(NS * RPT, TAIL)

    plsc.subcore_barrier()
    idx_src.wait()
    idx_dst.wait()

    pltpu.async_copy(featc.at[src_v.at[0]], rows.at[0], sem_g.at[0])
    pltpu.async_copy(featc.at[src_v.at[1]], rows.at[1], sem_g.at[1])

    def chunk_body(k, carry):
        b = lax.rem(k, 4)
        fb = lax.rem(k + 2, 4)

        @pl.when(k >= 2)
        def _():
            pltpu.make_async_copy(rows.at[fb], acc.at[dst_v.at[k - 2]],
                                  sem_s.at[fb]).wait()

        @pl.when(k + 2 < NCHUNK)
        def _():
            pltpu.async_copy(featc.at[src_v.at[k + 2]], rows.at[fb],
                             sem_g.at[fb])

        pltpu.make_async_copy(featc.at[src_v.at[k]], rows.at[b],
                              sem_g.at[b]).wait()
        pltpu.async_copy(rows.at[b], acc.at[dst_v.at[k]], sem_s.at[b],
                         add=True)
        return carry

    lax.fori_loop(0, NCHUNK, chunk_body, 0)
    for j in (NCHUNK - 2, NCHUNK - 1):
        pltpu.make_async_copy(rows.at[j % 4], acc.at[dst_v.at[j]],
                              sem_s.at[j % 4]).wait()
    plsc.subcore_barrier()

    pltpu.sync_copy(acc.at[pl.ds(s * RPT, RPT)],
                    out.at[pl.ds(s * RPT, RPT), pl.ds(c * H, H)])

    @pl.when(s == NS - 1)
    def _():
        pltpu.sync_copy(acc.at[pl.ds(NS * RPT, TAIL)],
                        out.at[pl.ds(NS * RPT, TAIL), pl.ds(c * H, H)])


@jax.jit
def kernel(feat, edge_index, eps):
    src = edge_index[0]
    dst = edge_index[1]

    featc = jnp.concatenate([feat[:, :H], feat[:, H:]], axis=0)

    pad = EPT_PAD - EPT
    src_t = jnp.pad(src.reshape(NS, EPT), ((0, 0), (0, pad)))
    src_t = src_t.reshape(NS, NCHUNK, CHUNK)
    srcp = jnp.stack([src_t, src_t + N_NODES])
    dstp = jnp.pad(dst.reshape(NS, EPT), ((0, 0), (0, pad)),
                   constant_values=TRASH).reshape(NS, NCHUNK, CHUNK)

    eps16 = jnp.broadcast_to(eps, (16,))

    mesh = plsc.VectorSubcoreMesh(core_axis_name="c", subcore_axis_name="s")
    out = pl.kernel(
        _gin_body,
        out_type=jax.ShapeDtypeStruct((N_NODES, D_FEAT), jnp.float32),
        mesh=mesh,
        compiler_params=pltpu.CompilerParams(use_tc_tiling_on_sc=False),
        scratch_types=[
            pltpu.VMEM_SHARED((N_PAD, H), jnp.float32),
            pltpu.VMEM((NCHUNK, CHUNK), jnp.int32),
            pltpu.VMEM((NCHUNK, CHUNK), jnp.int32),
            pltpu.VMEM((4, CHUNK, H), jnp.float32),
            pltpu.VMEM((IB, H), jnp.float32),
            pltpu.VMEM((16,), jnp.float32),
            pltpu.SemaphoreType.DMA((4,)),
            pltpu.SemaphoreType.DMA((4,)),
            pltpu.SemaphoreType.DMA,
        ],
    )(featc, srcp, dstp, eps16)
    return out


# speedup vs baseline: 1.3156x; 1.0174x over previous
import jax
import jax.numpy as jnp
from jax import lax
from jax.experimental import pallas as pl
from jax.experimental.pallas import tpu as pltpu
from jax.experimental.pallas import tpu_sc as plsc

N_NODES = 10000
N_EDGES = 320000
D_FEAT = 128
H = D_FEAT // 2
NC = 2
NS = 16
EPT = N_EDGES // NS
CHUNK = 128
NCHUNK = 158
EPT_PAD = NCHUNK * CHUNK
N_PAD = 10016
TRASH = N_NODES + 8
RPT = 624
TAIL = N_NODES - NS * RPT
IB = 104
NB = 5


def _gin_body(featc, srcp, dstp, eps16, out, acc, src_v, dst_v, rows,
              init_buf, eps_v, sem_g, sem_s, sem_i):
    c = lax.axis_index("c")
    s = lax.axis_index("s")

    idx_src = pltpu.async_copy(srcp.at[c, s], src_v, sem_i)
    idx_dst = pltpu.async_copy(dstp.at[s], dst_v, sem_i)

    pltpu.sync_copy(eps16, eps_v)
    scale = eps_v[...] + 1.0

    def init_range(r0, nrows):
        pltpu.sync_copy(featc.at[pl.ds(c * N_NODES + r0, nrows)],
                        init_buf.at[pl.ds(0, nrows)])

        def row_scale(r, carry):
            for j in range(H // 16):
                init_buf[r, pl.ds(j * 16, 16)] = (
                    init_buf[r, pl.ds(j * 16, 16)] * scale)
            return carry

        lax.fori_loop(0, nrows, row_scale, 0)
        pltpu.sync_copy(init_buf.at[pl.ds(0, nrows)],
                        acc.at[pl.ds(r0, nrows)])

    for p in range(RPT // IB):
        init_range(s * RPT + p * IB, IB)

    @pl.when(s == NS - 1)
    def _():
        init_range(NS * RPT, TAIL)

    plsc.subcore_barrier()
    idx_src.wait()
    idx_dst.wait()

    LOOK = NB - 2
    for j in range(LOOK):
        pltpu.async_copy(featc.at[src_v.at[j]], rows.at[j], sem_g.at[j])

    def chunk_body(k, carry):
        b = lax.rem(k, NB)
        fb = lax.rem(k + LOOK, NB)

        @pl.when(k >= 2)
        def _():
            pltpu.make_async_copy(rows.at[fb], acc.at[dst_v.at[k - 2]],
                                  sem_s.at[fb]).wait()

        @pl.when(k + LOOK < NCHUNK)
        def _():
            pltpu.async_copy(featc.at[src_v.at[k + LOOK]], rows.at[fb],
                             sem_g.at[fb])

        pltpu.make_async_copy(featc.at[src_v.at[k]], rows.at[b],
                              sem_g.at[b]).wait()
        pltpu.async_copy(rows.at[b], acc.at[dst_v.at[k]], sem_s.at[b],
                         add=True)
        return carry

    lax.fori_loop(0, NCHUNK, chunk_body, 0)
    for j in (NCHUNK - 2, NCHUNK - 1):
        pltpu.make_async_copy(rows.at[j % NB], acc.at[dst_v.at[j]],
                              sem_s.at[j % NB]).wait()
    plsc.subcore_barrier()

    pltpu.sync_copy(acc.at[pl.ds(s * RPT, RPT)],
                    out.at[pl.ds(s * RPT, RPT), pl.ds(c * H, H)])

    @pl.when(s == NS - 1)
    def _():
        pltpu.sync_copy(acc.at[pl.ds(NS * RPT, TAIL)],
                        out.at[pl.ds(NS * RPT, TAIL), pl.ds(c * H, H)])


@jax.jit
def kernel(feat, edge_index, eps):
    src = edge_index[0]
    dst = edge_index[1]

    featc = jnp.concatenate([feat[:, :H], feat[:, H:]], axis=0)

    pad = EPT_PAD - EPT
    src_t = jnp.pad(src.reshape(NS, EPT), ((0, 0), (0, pad)))
    src_t = src_t.reshape(NS, NCHUNK, CHUNK)
    srcp = jnp.stack([src_t, src_t + N_NODES])
    dstp = jnp.pad(dst.reshape(NS, EPT), ((0, 0), (0, pad)),
                   constant_values=TRASH).reshape(NS, NCHUNK, CHUNK)

    eps16 = jnp.broadcast_to(eps, (16,))

    mesh = plsc.VectorSubcoreMesh(core_axis_name="c", subcore_axis_name="s")
    out = pl.kernel(
        _gin_body,
        out_type=jax.ShapeDtypeStruct((N_NODES, D_FEAT), jnp.float32),
        mesh=mesh,
        compiler_params=pltpu.CompilerParams(use_tc_tiling_on_sc=False),
        scratch_types=[
            pltpu.VMEM_SHARED((N_PAD, H), jnp.float32),
            pltpu.VMEM((NCHUNK, CHUNK), jnp.int32),
            pltpu.VMEM((NCHUNK, CHUNK), jnp.int32),
            pltpu.VMEM((NB, CHUNK, H), jnp.float32),
            pltpu.VMEM((IB, H), jnp.float32),
            pltpu.VMEM((16,), jnp.float32),
            pltpu.SemaphoreType.DMA((NB,)),
            pltpu.SemaphoreType.DMA((NB,)),
            pltpu.SemaphoreType.DMA,
        ],
    )(featc, srcp, dstp, eps16)
    return out


# speedup vs baseline: 1.6047x; 1.2198x over previous
import jax
import jax.numpy as jnp
from jax import lax
from jax.experimental import pallas as pl
from jax.experimental.pallas import tpu as pltpu
from jax.experimental.pallas import tpu_sc as plsc

N_NODES = 10000
N_EDGES = 320000
D_FEAT = 128
H = D_FEAT // 2
NC = 2
NS = 16
EPT = N_EDGES // NS
CHUNK = 128
NCHUNK = 158
HCHUNK = NCHUNK // 2
EPT_PAD = NCHUNK * CHUNK
N_PAD = 10008
TRASH = N_NODES
RPT = 624
TAIL = N_NODES - NS * RPT
IB = 104
NB = 3


def _gin_body(featc, srcp, dstp, eps16, out, table, acc, src_v, dst_v, rows,
              eps_v, sem_g, sem_s, sem_i, sem_t):
    c = lax.axis_index("c")
    s = lax.axis_index("s")

    tbl = pltpu.async_copy(featc.at[pl.ds(c * N_NODES + s * RPT, RPT)],
                           table.at[pl.ds(s * RPT, RPT)], sem_t)
    idx_src = pltpu.async_copy(srcp.at[s, pl.ds(0, HCHUNK)], src_v, sem_i)
    idx_dst = pltpu.async_copy(dstp.at[s, pl.ds(0, HCHUNK)], dst_v, sem_i)

    pltpu.sync_copy(eps16, eps_v)
    scale = eps_v[...] + 1.0

    def init_range(r0, nrows):
        pltpu.sync_copy(featc.at[pl.ds(c * N_NODES + r0, nrows)],
                        rows.at[NB - 1, pl.ds(0, nrows)])

        def row_scale(r, carry):
            for j in range(H // 16):
                rows[NB - 1, r, pl.ds(j * 16, 16)] = (
                    rows[NB - 1, r, pl.ds(j * 16, 16)] * scale)
            return carry

        lax.fori_loop(0, nrows, row_scale, 0)
        pltpu.sync_copy(rows.at[NB - 1, pl.ds(0, nrows)],
                        acc.at[pl.ds(r0, nrows)])

    for p in range(RPT // IB):
        init_range(s * RPT + p * IB, IB)

    @pl.when(s == NS - 1)
    def _():
        init_range(NS * RPT, TAIL)

    tbl.wait()
    @pl.when(s == NS - 1)
    def _():
        pltpu.sync_copy(featc.at[pl.ds(c * N_NODES + NS * RPT, TAIL)],
                        table.at[pl.ds(NS * RPT, TAIL)])

    plsc.subcore_barrier()
    idx_src.wait()
    idx_dst.wait()

    def run_half():
        pltpu.async_copy(table.at[src_v.at[0]], rows.at[0], sem_g.at[0])

        def chunk_body(k, carry):
            b = lax.rem(k, NB)
            fb = lax.rem(k + 1, NB)

            @pl.when(k >= 2)
            def _():
                pltpu.make_async_copy(rows.at[fb], acc.at[dst_v.at[k - 2]],
                                      sem_s.at[fb]).wait()

            @pl.when(k + 1 < HCHUNK)
            def _():
                pltpu.async_copy(table.at[src_v.at[k + 1]], rows.at[fb],
                                 sem_g.at[fb])

            pltpu.make_async_copy(table.at[src_v.at[k]], rows.at[b],
                                  sem_g.at[b]).wait()
            pltpu.async_copy(rows.at[b], acc.at[dst_v.at[k]], sem_s.at[b],
                             add=True)
            return carry

        lax.fori_loop(0, HCHUNK, chunk_body, 0)
        for j in (HCHUNK - 2, HCHUNK - 1):
            pltpu.make_async_copy(rows.at[j % NB], acc.at[dst_v.at[j]],
                                  sem_s.at[j % NB]).wait()

    run_half()
    pltpu.sync_copy(srcp.at[s, pl.ds(HCHUNK, HCHUNK)], src_v)
    pltpu.sync_copy(dstp.at[s, pl.ds(HCHUNK, HCHUNK)], dst_v)
    run_half()
    plsc.subcore_barrier()

    pltpu.sync_copy(acc.at[pl.ds(s * RPT, RPT)],
                    out.at[pl.ds(s * RPT, RPT), pl.ds(c * H, H)])

    @pl.when(s == NS - 1)
    def _():
        pltpu.sync_copy(acc.at[pl.ds(NS * RPT, TAIL)],
                        out.at[pl.ds(NS * RPT, TAIL), pl.ds(c * H, H)])


@jax.jit
def kernel(feat, edge_index, eps):
    src = edge_index[0]
    dst = edge_index[1]

    featc = jnp.concatenate([feat[:, :H], feat[:, H:]], axis=0)

    pad = EPT_PAD - EPT
    srcp = jnp.pad(src.reshape(NS, EPT), ((0, 0), (0, pad)))
    srcp = srcp.reshape(NS, NCHUNK, CHUNK)
    dstp = jnp.pad(dst.reshape(NS, EPT), ((0, 0), (0, pad)),
                   constant_values=TRASH).reshape(NS, NCHUNK, CHUNK)

    eps16 = jnp.broadcast_to(eps, (16,))

    mesh = plsc.VectorSubcoreMesh(core_axis_name="c", subcore_axis_name="s")
    out = pl.kernel(
        _gin_body,
        out_type=jax.ShapeDtypeStruct((N_NODES, D_FEAT), jnp.float32),
        mesh=mesh,
        compiler_params=pltpu.CompilerParams(use_tc_tiling_on_sc=False),
        scratch_types=[
            pltpu.VMEM_SHARED((N_NODES, H), jnp.float32),
            pltpu.VMEM_SHARED((N_PAD, H), jnp.float32),
            pltpu.VMEM((HCHUNK, CHUNK), jnp.int32),
            pltpu.VMEM((HCHUNK, CHUNK), jnp.int32),
            pltpu.VMEM((NB, CHUNK, H), jnp.float32),
            pltpu.VMEM((16,), jnp.float32),
            pltpu.SemaphoreType.DMA((NB,)),
            pltpu.SemaphoreType.DMA((NB,)),
            pltpu.SemaphoreType.DMA,
            pltpu.SemaphoreType.DMA,
        ],
    )(featc, srcp, dstp, eps16)
    return out


# speedup vs baseline: 1.8404x; 1.1468x over previous
import jax
import jax.numpy as jnp
from jax import lax
from jax.experimental import pallas as pl
from jax.experimental.pallas import tpu as pltpu
from jax.experimental.pallas import tpu_sc as plsc

N_NODES = 10000
N_EDGES = 320000
D_FEAT = 128
H = D_FEAT // 2
NC = 2
NS = 16
EPT = N_EDGES // NS
CHUNK = 128
NCHUNK = 158
HCHUNK = NCHUNK // 2
EPT_PAD = NCHUNK * CHUNK
N_PAD = 10008
TRASH = N_NODES
RPT = 624
TAIL = N_NODES - NS * RPT
IB = 104
NB = 3


def _gin_body(feat, srcp, dstp, eps16, out, table, acc, src_v, dst_v, rows,
              eps_v, sem_g, sem_s, sem_i, sem_t):
    c = lax.axis_index("c")
    s = lax.axis_index("s")

    tbl = pltpu.async_copy(feat.at[pl.ds(s * RPT, RPT), pl.ds(c * H, H)],
                           table.at[pl.ds(s * RPT, RPT)], sem_t)
    idx_src = pltpu.async_copy(srcp.at[s, pl.ds(0, HCHUNK)], src_v, sem_i)
    idx_dst = pltpu.async_copy(dstp.at[s, pl.ds(0, HCHUNK)], dst_v, sem_i)

    pltpu.sync_copy(eps16, eps_v)
    scale = eps_v[...] + 1.0

    @pl.when(s == NS - 1)
    def _():
        pltpu.sync_copy(feat.at[pl.ds(NS * RPT, TAIL), pl.ds(c * H, H)],
                        table.at[pl.ds(NS * RPT, TAIL)])

    tbl.wait()

    def init_range(r0, nrows):
        pltpu.sync_copy(table.at[pl.ds(r0, nrows)],
                        rows.at[NB - 1, pl.ds(0, nrows)])

        def row_scale(r, carry):
            for j in range(H // 16):
                rows[NB - 1, r, pl.ds(j * 16, 16)] = (
                    rows[NB - 1, r, pl.ds(j * 16, 16)] * scale)
            return carry

        lax.fori_loop(0, nrows, row_scale, 0)
        pltpu.sync_copy(rows.at[NB - 1, pl.ds(0, nrows)],
                        acc.at[pl.ds(r0, nrows)])

    for p in range(RPT // IB):
        init_range(s * RPT + p * IB, IB)

    @pl.when(s == NS - 1)
    def _():
        init_range(NS * RPT, TAIL)

    plsc.subcore_barrier()
    idx_src.wait()
    idx_dst.wait()

    def run_half():
        pltpu.async_copy(table.at[src_v.at[0]], rows.at[0], sem_g.at[0])

        def chunk_body(k, carry):
            b = lax.rem(k, NB)
            fb = lax.rem(k + 1, NB)

            @pl.when(k >= 2)
            def _():
                pltpu.make_async_copy(rows.at[fb], acc.at[dst_v.at[k - 2]],
                                      sem_s.at[fb]).wait()

            @pl.when(k + 1 < HCHUNK)
            def _():
                pltpu.async_copy(table.at[src_v.at[k + 1]], rows.at[fb],
                                 sem_g.at[fb])

            pltpu.make_async_copy(table.at[src_v.at[k]], rows.at[b],
                                  sem_g.at[b]).wait()
            pltpu.async_copy(rows.at[b], acc.at[dst_v.at[k]], sem_s.at[b],
                             add=True)
            return carry

        lax.fori_loop(0, HCHUNK, chunk_body, 0)
        for j in (HCHUNK - 2, HCHUNK - 1):
            pltpu.make_async_copy(rows.at[j % NB], acc.at[dst_v.at[j]],
                                  sem_s.at[j % NB]).wait()

    run_half()
    pltpu.sync_copy(srcp.at[s, pl.ds(HCHUNK, HCHUNK)], src_v)
    pltpu.sync_copy(dstp.at[s, pl.ds(HCHUNK, HCHUNK)], dst_v)
    run_half()
    plsc.subcore_barrier()

    pltpu.sync_copy(acc.at[pl.ds(s * RPT, RPT)],
                    out.at[pl.ds(s * RPT, RPT), pl.ds(c * H, H)])

    @pl.when(s == NS - 1)
    def _():
        pltpu.sync_copy(acc.at[pl.ds(NS * RPT, TAIL)],
                        out.at[pl.ds(NS * RPT, TAIL), pl.ds(c * H, H)])


@jax.jit
def kernel(feat, edge_index, eps):
    src = edge_index[0]
    dst = edge_index[1]

    pad = EPT_PAD - EPT
    srcp = jnp.pad(src.reshape(NS, EPT), ((0, 0), (0, pad)))
    srcp = srcp.reshape(NS, NCHUNK, CHUNK)
    dstp = jnp.pad(dst.reshape(NS, EPT), ((0, 0), (0, pad)),
                   constant_values=TRASH).reshape(NS, NCHUNK, CHUNK)

    eps16 = jnp.broadcast_to(eps, (16,))

    mesh = plsc.VectorSubcoreMesh(core_axis_name="c", subcore_axis_name="s")
    out = pl.kernel(
        _gin_body,
        out_type=jax.ShapeDtypeStruct((N_NODES, D_FEAT), jnp.float32),
        mesh=mesh,
        compiler_params=pltpu.CompilerParams(use_tc_tiling_on_sc=False),
        scratch_types=[
            pltpu.VMEM_SHARED((N_NODES, H), jnp.float32),
            pltpu.VMEM_SHARED((N_PAD, H), jnp.float32),
            pltpu.VMEM((HCHUNK, CHUNK), jnp.int32),
            pltpu.VMEM((HCHUNK, CHUNK), jnp.int32),
            pltpu.VMEM((NB, CHUNK, H), jnp.float32),
            pltpu.VMEM((16,), jnp.float32),
            pltpu.SemaphoreType.DMA((NB,)),
            pltpu.SemaphoreType.DMA((NB,)),
            pltpu.SemaphoreType.DMA,
            pltpu.SemaphoreType.DMA,
        ],
    )(feat, srcp, dstp, eps16)
    return out
